# baseline (speedup 1.0000x reference)
import functools

import jax
import jax.numpy as jnp
from jax import lax
from jax.experimental import pallas as pl


def _ce_body(tgt_ref, x_ref, out_ref, *, num_tokens, nblocks, vocab, chunk,
             rows):
    nchunks = vocab // chunk
    tb = x_ref.shape[0]

    lane = jax.lax.broadcasted_iota(jnp.int32, (rows, chunk), 1)

    part = jnp.zeros((1, 1), jnp.float32)
    for g in range(tb // rows):
        r0 = g * rows
        t_b = tgt_ref[r0:r0 + rows, :]

        m_l = jnp.full((rows, chunk), -jnp.inf, jnp.float32)
        for c in range(nchunks):
            m_l = jnp.maximum(
                m_l, x_ref[r0:r0 + rows, c * chunk:(c + 1) * chunk])
        m_row = jnp.max(m_l, axis=-1, keepdims=True)

        s_l = jnp.zeros((rows, chunk), jnp.float32)
        tgt_l = jnp.zeros((rows, chunk), jnp.float32)
        for c in range(nchunks):
            d = x_ref[r0:r0 + rows, c * chunk:(c + 1) * chunk] - m_row
            eq = (lane + c * chunk) == t_b
            s_l = s_l + jnp.exp(d)
            tgt_l = tgt_l + jnp.where(eq, d, 0.0)
        s_row = jnp.sum(s_l, axis=-1, keepdims=True)
        tgt_row = jnp.sum(tgt_l, axis=-1, keepdims=True) + m_row

        lse = m_row + jnp.log(s_row)
        part = part + jnp.sum(lse - tgt_row, keepdims=True).reshape(1, 1)

    i = pl.program_id(0)

    @pl.when(i == 0)
    def _init():
        out_ref[...] = jnp.zeros((1, 1), jnp.float32)

    out_ref[...] += part

    @pl.when(i == nblocks - 1)
    def _fin():
        out_ref[...] = out_ref[...] * (1.0 / num_tokens)


@functools.partial(jax.jit, static_argnames=("block_tokens", "chunk", "rows"))
def _ce_mean(logits_flat, targets_col, block_tokens, chunk, rows):
    num_tokens, vocab = logits_flat.shape
    nblocks = num_tokens // block_tokens
    body = functools.partial(
        _ce_body, num_tokens=num_tokens, nblocks=nblocks, vocab=vocab,
        chunk=chunk, rows=rows,
    )
    out = pl.pallas_call(
        body,
        grid=(nblocks,),
        in_specs=[
            pl.BlockSpec((block_tokens, 1), lambda i: (i, 0)),
            pl.BlockSpec((block_tokens, vocab), lambda i: (i, 0)),
        ],
        out_specs=pl.BlockSpec((1, 1), lambda i: (0, 0)),
        out_shape=jax.ShapeDtypeStruct((1, 1), jnp.float32),
    )(targets_col, logits_flat)
    return out[0, 0]


def kernel(logits, targets):
    vocab = logits.shape[-1]
    logits_flat = logits.reshape(-1, vocab)
    targets_col = targets.reshape(-1, 1).astype(jnp.int32)
    return _ce_mean(logits_flat, targets_col, 128, 128, 128)


# speedup vs baseline: 1.1273x; 1.1273x over previous
import functools

import jax
import jax.numpy as jnp
from jax import lax
from jax.experimental import pallas as pl
from jax.experimental.pallas import tpu as pltpu


def _ce_body(tgt_ref, x_ref, out_ref, *, num_tokens, nblocks, chunk):
    tb = x_ref.shape[0]
    x = x_ref[...]
    m_row = jnp.max(x, axis=-1, keepdims=True)
    s_row = jnp.sum(jnp.exp(x - m_row), axis=-1, keepdims=True)
    lse = m_row + jnp.log(s_row)

    lane = jax.lax.broadcasted_iota(jnp.int32, (1, chunk), 1)
    acc = jnp.zeros((1, chunk), jnp.float32)
    for r in range(tb):
        t = tgt_ref[r, 0]
        off = pl.multiple_of((t >> 7) << 7, chunk)
        v = x_ref[pl.ds(r, 1), pl.ds(off, chunk)]
        acc = acc + jnp.where(lane == (t & (chunk - 1)), v, 0.0)
    tgt_total = jnp.sum(acc, keepdims=True).reshape(1, 1)

    part = jnp.sum(lse, keepdims=True).reshape(1, 1) - tgt_total

    i = pl.program_id(0)

    @pl.when(i == 0)
    def _init():
        out_ref[...] = jnp.zeros((1, 1), jnp.float32)

    out_ref[...] += part

    @pl.when(i == nblocks - 1)
    def _fin():
        out_ref[...] = out_ref[...] * (1.0 / num_tokens)


@functools.partial(jax.jit, static_argnames=("block_tokens",))
def _ce_mean(logits_flat, targets_col, block_tokens):
    num_tokens, vocab = logits_flat.shape
    nblocks = num_tokens // block_tokens
    body = functools.partial(
        _ce_body, num_tokens=num_tokens, nblocks=nblocks, chunk=128,
    )
    out = pl.pallas_call(
        body,
        grid=(nblocks,),
        in_specs=[
            pl.BlockSpec((block_tokens, 1), lambda i: (i, 0),
                         memory_space=pltpu.SMEM),
            pl.BlockSpec((block_tokens, vocab), lambda i: (i, 0)),
        ],
        out_specs=pl.BlockSpec((1, 1), lambda i: (0, 0)),
        out_shape=jax.ShapeDtypeStruct((1, 1), jnp.float32),
    )(targets_col, logits_flat)
    return out[0, 0]


def kernel(logits, targets):
    vocab = logits.shape[-1]
    logits_flat = logits.reshape(-1, vocab)
    targets_col = targets.reshape(-1, 1).astype(jnp.int32)
    return _ce_mean(logits_flat, targets_col, 128)
